# baseline (speedup 1.0000x reference)
import jax
import jax.numpy as jnp
from jax.experimental import pallas as pl
from jax.experimental.pallas import tpu as pltpu

BS, L, DIM = 4, 512, 128
IT = 128
NIT = L // IT


def _gcn_body(w2_ref, q_ref, qi_ref, qT_ref, dep_ref, a_ref, bT_ref,
              wd_ref, out_ref, prejT_ref):
    qi = qi_ref[0]
    qT = qT_ref[0]
    pre_i = jnp.dot(qi, a_ref[...], preferred_element_type=jnp.float32)
    prejT_ref[...] = jnp.dot(bT_ref[...], qT,
                             preferred_element_type=jnp.float32)

    d_iota = jax.lax.broadcasted_iota(jnp.int32, (DIM, 1), 0)

    def d_step(d, acc):
        onehot = (d_iota == d).astype(jnp.float32)
        col = jnp.dot(pre_i, onehot,
                      preferred_element_type=jnp.float32)
        row = prejT_ref[pl.ds(d, 1), :]
        return acc + jnp.maximum(col + row, 0.0) * w2_ref[d]

    t = jax.lax.fori_loop(0, DIM, d_step,
                          jnp.zeros((IT, L), jnp.float32))

    mask = dep_ref[0] > 0
    T = jnp.where(mask, t, jnp.float32(-100.0))
    m = jnp.max(T, axis=1, keepdims=True)
    e = jnp.exp(T - m)
    beta = e / jnp.sum(e, axis=1, keepdims=True)
    betam = beta * mask.astype(jnp.float32)

    Hj = jnp.dot(q_ref[0], wd_ref[...],
                 preferred_element_type=jnp.float32)
    agg = jnp.dot(betam, Hj, preferred_element_type=jnp.float32)
    out_ref[0] = jnp.maximum(qi + agg, 0.0)


def kernel(queries, wordlens, syntactic_dep, W1, W2, Wd):
    q = queries.astype(jnp.float32)
    qT = jnp.swapaxes(q, 1, 2)
    dep = syntactic_dep.astype(jnp.int32)
    A = W1[:DIM, :]
    BT = jnp.swapaxes(W1[DIM:, :], 0, 1)
    w2 = W2[:, 0]

    grid = (BS, NIT)
    out = pl.pallas_call(
        _gcn_body,
        grid_spec=pltpu.PrefetchScalarGridSpec(
            num_scalar_prefetch=1,
            grid=grid,
            in_specs=[
                pl.BlockSpec((1, L, DIM), lambda b, it, w2: (b, 0, 0)),
                pl.BlockSpec((1, IT, DIM), lambda b, it, w2: (b, it, 0)),
                pl.BlockSpec((1, DIM, L), lambda b, it, w2: (b, 0, 0)),
                pl.BlockSpec((1, IT, L), lambda b, it, w2: (b, it, 0)),
                pl.BlockSpec((DIM, DIM), lambda b, it, w2: (0, 0)),
                pl.BlockSpec((DIM, DIM), lambda b, it, w2: (0, 0)),
                pl.BlockSpec((DIM, DIM), lambda b, it, w2: (0, 0)),
            ],
            out_specs=pl.BlockSpec((1, IT, DIM), lambda b, it, w2: (b, it, 0)),
            scratch_shapes=[pltpu.VMEM((DIM, L), jnp.float32)],
        ),
        out_shape=jax.ShapeDtypeStruct((BS, L, DIM), jnp.float32),
        compiler_params=pltpu.CompilerParams(
            dimension_semantics=("arbitrary", "arbitrary"),
        ),
    )(w2, q, q, qT, dep, A, BT, Wd)

    return (out, wordlens, syntactic_dep)


# speedup vs baseline: 2.9299x; 2.9299x over previous
import jax
import jax.numpy as jnp
from jax.experimental import pallas as pl
from jax.experimental.pallas import tpu as pltpu

BS, L, DIM = 4, 512, 128
IG = 128
NG = L // IG
CH = 8
NCH = IG // CH


def _gcn_body(q_ref, qg_ref, qT_ref, qgT_ref, depT_ref, a_ref, b_ref, w2bd_ref,
              wdT_ref, outT_ref, prei_ref, prej_ref, h8_ref, gacc_ref):
    prei_ref[...] = jnp.dot(qg_ref[0], a_ref[...],
                            preferred_element_type=jnp.float32)
    prej_ref[...] = jnp.dot(q_ref[0], b_ref[...],
                            preferred_element_type=jnp.float32)
    gacc_ref[...] = jnp.zeros((L, IG), jnp.float32)

    u_iota = jax.lax.broadcasted_iota(jnp.int32, (CH, IG), 0)
    l_iota = jax.lax.broadcasted_iota(jnp.int32, (CH, IG), 1)

    def chunk(k, _):
        buf = jax.lax.rem(k, 2)
        prej = prej_ref[...]
        for u in range(CH):
            r = prei_ref[pl.ds(k * CH + u, 1), :]
            h8_ref[buf, :, DIM * u:DIM * (u + 1)] = (
                jnp.maximum(prej + r, 0.0))
        tmp = jnp.dot(h8_ref[buf], w2bd_ref[...],
                      preferred_element_type=jnp.float32)
        place = (l_iota == CH * k + u_iota).astype(jnp.float32)
        gacc_ref[...] += jnp.dot(tmp, place,
                                 preferred_element_type=jnp.float32)
        return 0

    jax.lax.fori_loop(0, NCH, chunk, 0)

    maskT = depT_ref[0] > 0
    T = jnp.where(maskT, gacc_ref[...], jnp.float32(-100.0))
    m = jnp.max(T, axis=0, keepdims=True)
    e = jnp.exp(T - m)
    betam = e / jnp.sum(e, axis=0, keepdims=True) * maskT.astype(jnp.float32)

    HjT = jnp.dot(wdT_ref[...], qT_ref[0],
                  preferred_element_type=jnp.float32)
    aggT = jnp.dot(HjT, betam, preferred_element_type=jnp.float32)
    outT_ref[0] = jnp.maximum(qgT_ref[0] + aggT, 0.0)


def kernel(queries, wordlens, syntactic_dep, W1, W2, Wd):
    q = queries.astype(jnp.float32)
    qT = jnp.swapaxes(q, 1, 2)
    depT = jnp.swapaxes(syntactic_dep.astype(jnp.int32), 1, 2)
    A = W1[:DIM, :]
    B = W1[DIM:, :]
    W2bd = jnp.kron(jnp.eye(CH, dtype=jnp.float32), W2)
    WdT = jnp.swapaxes(Wd, 0, 1)

    outT = pl.pallas_call(
        _gcn_body,
        grid=(BS, NG),
        in_specs=[
            pl.BlockSpec((1, L, DIM), lambda b, g: (b, 0, 0)),
            pl.BlockSpec((1, IG, DIM), lambda b, g: (b, g, 0)),
            pl.BlockSpec((1, DIM, L), lambda b, g: (b, 0, 0)),
            pl.BlockSpec((1, DIM, IG), lambda b, g: (b, 0, g)),
            pl.BlockSpec((1, L, IG), lambda b, g: (b, 0, g)),
            pl.BlockSpec((DIM, DIM), lambda b, g: (0, 0)),
            pl.BlockSpec((DIM, DIM), lambda b, g: (0, 0)),
            pl.BlockSpec((CH * DIM, CH), lambda b, g: (0, 0)),
            pl.BlockSpec((DIM, DIM), lambda b, g: (0, 0)),
        ],
        out_specs=pl.BlockSpec((1, DIM, IG), lambda b, g: (b, 0, g)),
        out_shape=jax.ShapeDtypeStruct((BS, DIM, L), jnp.float32),
        scratch_shapes=[
            pltpu.VMEM((IG, DIM), jnp.float32),
            pltpu.VMEM((L, DIM), jnp.float32),
            pltpu.VMEM((2, L, CH * DIM), jnp.float32),
            pltpu.VMEM((L, IG), jnp.float32),
        ],
        compiler_params=pltpu.CompilerParams(
            dimension_semantics=("arbitrary", "arbitrary"),
        ),
    )(q, q, qT, qT, depT, A, B, W2bd, WdT)

    out = jnp.swapaxes(outT, 1, 2)
    return (out, wordlens, syntactic_dep)
